# baseline (speedup 1.0000x reference)
import functools

import jax
import jax.numpy as jnp
from jax import lax
from jax.experimental import pallas as pl
from jax.experimental.pallas import tpu as pltpu
from jax.experimental.pallas import tpu_sc as plsc


def _make_gather(nw, nc, seq, bpw, D, V, bsz):
    ns = nw // nc
    v_per_tile = V // ns
    mesh = plsc.VectorSubcoreMesh(core_axis_name="c", subcore_axis_name="s")

    @functools.partial(
        pl.kernel,
        mesh=mesh,
        out_type=jax.ShapeDtypeStruct((seq, bsz, D), jnp.float32),
        scratch_types=[
            pltpu.VMEM((seq, bpw), jnp.int32),
            pltpu.VMEM((3, 2, bpw, D), jnp.float32),
            pltpu.VMEM_SHARED((V, D), jnp.float32),
            pltpu.SemaphoreType.DMA((6,)),
            pltpu.SemaphoreType.DMA((3,)),
        ],
    )
    def k(idx_hbm, table_hbm, out_hbm, idx_v, rows_v, table_sh, gsem, wsem):
        sid = lax.axis_index("s")
        wid = sid * nc + lax.axis_index("c")
        base = wid * bpw
        nch = seq // 2
        tstage = pltpu.make_async_copy(
            table_hbm.at[pl.ds(sid * v_per_tile, v_per_tile)],
            table_sh.at[pl.ds(sid * v_per_tile, v_per_tile)], gsem.at[4])
        istage = pltpu.make_async_copy(
            idx_hbm.at[:, pl.ds(base, bpw)], idx_v, gsem.at[5])
        tstage.start()
        istage.start()
        istage.wait()
        tstage.wait()
        plsc.subcore_barrier()

        def gather(m, h, p):
            return pltpu.make_async_copy(
                table_sh.at[idx_v.at[2 * m + h]], rows_v.at[p, h],
                gsem.at[2 * p + h])

        def write(m, p):
            return pltpu.make_async_copy(
                rows_v.at[p], out_hbm.at[pl.ds(2 * m, 2), pl.ds(base, bpw)],
                wsem.at[p])

        gather(0, 0, 0).start()
        gather(0, 1, 0).start()

        def body(m, carry):
            p = m % 3
            q = (m + 1) % 3

            @pl.when(m >= 2)
            def _():
                write(m - 2, q).wait()

            @pl.when(m + 1 < nch)
            def _():
                gather(m + 1, 0, q).start()
                gather(m + 1, 1, q).start()

            gather(m, 0, p).wait()
            gather(m, 1, p).wait()
            write(m, p).start()
            return carry

        lax.fori_loop(0, nch, body, 0)
        write(nch - 2, (nch - 2) % 3).wait()
        write(nch - 1, (nch - 1) % 3).wait()

    return k


def kernel(indices, weights):
    bsz, seq = indices.shape
    V, D = weights.shape
    info = plsc.get_sparse_core_info()
    nc, ns = info.num_cores, info.num_subcores
    nw = nc * ns
    idxT = indices.astype(jnp.int32).T
    pad = (-bsz) % nw
    if pad:
        idxT = jnp.concatenate(
            [idxT, jnp.zeros((seq, pad), jnp.int32)], axis=1)
    bpw = (bsz + pad) // nw
    out = _make_gather(nw, nc, seq, bpw, D, V, bsz + pad)(idxT, weights)
    return jnp.transpose(out, (1, 0, 2))[:bsz]


# speedup vs baseline: 1.0019x; 1.0019x over previous
import functools

import jax
import jax.numpy as jnp
from jax import lax
from jax.experimental import pallas as pl
from jax.experimental.pallas import tpu as pltpu
from jax.experimental.pallas import tpu_sc as plsc


def _make_gather(nw, nc, seq, bpw, D, V, bsz):
    ns = nw // nc
    v_per_tile = V // ns
    mesh = plsc.VectorSubcoreMesh(core_axis_name="c", subcore_axis_name="s")

    @functools.partial(
        pl.kernel,
        mesh=mesh,
        out_type=jax.ShapeDtypeStruct((seq, bsz, D), jnp.float32),
        scratch_types=[
            pltpu.VMEM((seq, bpw), jnp.int32),
            pltpu.VMEM((6, bpw, D), jnp.float32),
            pltpu.VMEM_SHARED((V, D), jnp.float32),
            pltpu.SemaphoreType.DMA((6,)),
            pltpu.SemaphoreType.DMA((6,)),
        ],
    )
    def k(idx_hbm, table_hbm, out_hbm, idx_v, rows_v, table_sh, gsem, wsem):
        sid = lax.axis_index("s")
        wid = sid * nc + lax.axis_index("c")
        base = wid * bpw
        tstage = pltpu.make_async_copy(
            table_hbm.at[pl.ds(sid * v_per_tile, v_per_tile)],
            table_sh.at[pl.ds(sid * v_per_tile, v_per_tile)], gsem.at[2])
        istage = pltpu.make_async_copy(
            idx_hbm.at[:, pl.ds(base, bpw)], idx_v, gsem.at[3])
        tstage.start()
        istage.start()
        istage.wait()
        tstage.wait()
        plsc.subcore_barrier()

        def gather(j, p):
            return pltpu.make_async_copy(
                table_sh.at[idx_v.at[j]], rows_v.at[p], gsem.at[p])

        def write(j, p):
            return pltpu.make_async_copy(
                rows_v.at[p], out_hbm.at[j, pl.ds(base, bpw)], wsem.at[p])

        gather(0, 0).start()
        gather(1, 1).start()

        def body(j, carry):
            p = j % 6
            q = (j + 2) % 6

            @pl.when(j >= 4)
            def _():
                write(j - 4, q).wait()

            @pl.when(j + 2 < seq)
            def _():
                gather(j + 2, q).start()

            gather(j, p).wait()
            write(j, p).start()
            return carry

        lax.fori_loop(0, seq, body, 0)
        for t in range(4):
            j = seq - 4 + t
            write(j, j % 6).wait()

    return k


def kernel(indices, weights):
    bsz, seq = indices.shape
    V, D = weights.shape
    info = plsc.get_sparse_core_info()
    nc, ns = info.num_cores, info.num_subcores
    nw = nc * ns
    idxT = indices.astype(jnp.int32).T
    pad = (-bsz) % nw
    if pad:
        idxT = jnp.concatenate(
            [idxT, jnp.zeros((seq, pad), jnp.int32)], axis=1)
    bpw = (bsz + pad) // nw
    out = _make_gather(nw, nc, seq, bpw, D, V, bsz + pad)(idxT, weights)
    return jnp.transpose(out, (1, 0, 2))[:bsz]
